# Initial kernel scaffold; baseline (speedup 1.0000x reference)
import jax
import jax.numpy as jnp
from jax.experimental import pallas as pl


def kernel(x, emb_table, fc_w, fc_b):
    raise NotImplementedError("write your pallas kernel here")


# speedup vs baseline: 12.2727x; 12.2727x over previous
import functools

import jax
import jax.numpy as jnp
from jax import lax
from jax.experimental import pallas as pl
from jax.experimental.pallas import tpu as pltpu
from jax.experimental.pallas import tpu_sc as plsc

_B = 16384
_L = 200
_EMB = 64
_NLAB = 50
_VROWS = 100001

_NC = 2
_NS = 16
_NW = _NC * _NS
_SPW = _B // _NW
_HALF = _L // 2
_CH = 32

_mesh = plsc.VectorSubcoreMesh(
    core_axis_name="c", subcore_axis_name="s",
    num_cores=_NC, num_subcores=_NS)


@functools.partial(
    pl.kernel,
    out_type=jax.ShapeDtypeStruct((_B, _EMB), jnp.float32),
    mesh=_mesh,
    scratch_types=[
        pltpu.VMEM((_CH, 2, _HALF), jnp.int32),
        pltpu.VMEM((_L, _EMB), jnp.float32),
        pltpu.VMEM((_SPW, _EMB), jnp.float32),
        pltpu.SemaphoreType.DMA,
    ],
    compiler_params=pltpu.CompilerParams(use_tc_tiling_on_sc=False),
)
def _sc_pool(x_hbm, table_hbm, pooled_hbm, idx_v, rows_v, out_v, sem):
    wid = lax.axis_index("s") * _NC + lax.axis_index("c")
    base = wid * _SPW

    def chunk_body(c, carry):
        pltpu.sync_copy(x_hbm.at[pl.ds(base + c * _CH, _CH)], idx_v)

        def sample_body(s, carry2):
            cp0 = pltpu.async_copy(
                table_hbm.at[idx_v.at[s, 0]], rows_v.at[pl.ds(0, _HALF)], sem)
            cp1 = pltpu.async_copy(
                table_hbm.at[idx_v.at[s, 1]], rows_v.at[pl.ds(_HALF, _HALF)],
                sem)
            cp0.wait()
            cp1.wait()

            def acc_body(r, accs):
                a0, a1, a2, a3 = accs
                a0 = a0 + rows_v[r, pl.ds(0, 16)]
                a1 = a1 + rows_v[r, pl.ds(16, 16)]
                a2 = a2 + rows_v[r, pl.ds(32, 16)]
                a3 = a3 + rows_v[r, pl.ds(48, 16)]
                return (a0, a1, a2, a3)

            z = jnp.zeros((16,), jnp.float32)
            a0, a1, a2, a3 = lax.fori_loop(0, _L, acc_body, (z, z, z, z))
            row = c * _CH + s
            out_v[row, pl.ds(0, 16)] = a0
            out_v[row, pl.ds(16, 16)] = a1
            out_v[row, pl.ds(32, 16)] = a2
            out_v[row, pl.ds(48, 16)] = a3
            return carry2

        lax.fori_loop(0, _CH, sample_body, 0)
        return carry

    lax.fori_loop(0, _SPW // _CH, chunk_body, 0)
    pltpu.sync_copy(out_v, pooled_hbm.at[pl.ds(base, _SPW)])


def _mm_body(p_ref, w_ref, b_ref, o_ref):
    o_ref[...] = (
        jnp.dot(p_ref[...] * (1.0 / _L), w_ref[...],
                preferred_element_type=jnp.float32)
        + b_ref[...])


_mm = pl.pallas_call(
    _mm_body,
    out_shape=jax.ShapeDtypeStruct((_B, _NLAB), jnp.float32),
    grid=(8,),
    in_specs=[
        pl.BlockSpec((_B // 8, _EMB), lambda i: (i, 0)),
        pl.BlockSpec((_EMB, _NLAB), lambda i: (0, 0)),
        pl.BlockSpec((1, _NLAB), lambda i: (0, 0)),
    ],
    out_specs=pl.BlockSpec((_B // 8, _NLAB), lambda i: (i, 0)),
)


def kernel(x, emb_table, fc_w, fc_b):
    x3 = x.reshape(_B, 2, _HALF)
    pooled = _sc_pool(x3, emb_table)
    return _mm(pooled, fc_w.T, fc_b.reshape(1, _NLAB))


# speedup vs baseline: 26.3266x; 2.1451x over previous
import functools

import jax
import jax.numpy as jnp
from jax import lax
from jax.experimental import pallas as pl
from jax.experimental.pallas import tpu as pltpu
from jax.experimental.pallas import tpu_sc as plsc

_B = 16384
_L = 200
_EMB = 64
_NLAB = 50
_VROWS = 100001
_W32 = _EMB // 2

_NC = 2
_NS = 16
_NW = _NC * _NS
_SPW = _B // _NW
_HALF = _L // 2
_CH = 32
_NCHUNK = _SPW // _CH

_mesh = plsc.VectorSubcoreMesh(
    core_axis_name="c", subcore_axis_name="s",
    num_cores=_NC, num_subcores=_NS)


@functools.partial(
    pl.kernel,
    out_type=jax.ShapeDtypeStruct((_B, _EMB), jnp.float32),
    mesh=_mesh,
    scratch_types=[
        pltpu.VMEM((2, _CH, 2, _HALF), jnp.int32),
        pltpu.VMEM((2, _L, _W32), jnp.int32),
        pltpu.VMEM((_SPW, _EMB), jnp.float32),
        pltpu.SemaphoreType.DMA,
        pltpu.SemaphoreType.DMA,
    ],
    compiler_params=pltpu.CompilerParams(use_tc_tiling_on_sc=False,
                                         needs_layout_passes=False),
)
def _sc_pool(x_hbm, table_hbm, pooled_hbm, idx_v, rows_v, out_v,
             sem_g, sem_i):
    wid = lax.axis_index("s") * _NC + lax.axis_index("c")
    base = wid * _SPW

    def issue_sample(cbuf, s_local, rbuf):
        pltpu.async_copy(table_hbm.at[idx_v.at[cbuf, s_local, 0]],
                         rows_v.at[rbuf, pl.ds(0, _HALF)], sem_g)
        pltpu.async_copy(table_hbm.at[idx_v.at[cbuf, s_local, 1]],
                         rows_v.at[rbuf, pl.ds(_HALF, _HALF)], sem_g)

    def wait_sample(rbuf):
        pltpu.make_async_copy(table_hbm.at[pl.ds(0, _L)],
                              rows_v.at[rbuf], sem_g).wait()

    hi_mask = jnp.int32(-65536)

    def accumulate(rbuf, out_row):
        def acc_body(r, accs):
            a0, a1, a2, a3 = accs
            v0 = rows_v[rbuf, r, pl.ds(0, 16)]
            v1 = rows_v[rbuf, r, pl.ds(16, 16)]
            a0 = a0 + plsc.bitcast(v0 << 16, jnp.float32)
            a1 = a1 + plsc.bitcast(v0 & hi_mask, jnp.float32)
            a2 = a2 + plsc.bitcast(v1 << 16, jnp.float32)
            a3 = a3 + plsc.bitcast(v1 & hi_mask, jnp.float32)
            return (a0, a1, a2, a3)

        z = jnp.zeros((16,), jnp.float32)
        a0, a1, a2, a3 = lax.fori_loop(0, _L, acc_body, (z, z, z, z),
                                       unroll=4)
        out_v[out_row, pl.ds(0, 16)] = a0
        out_v[out_row, pl.ds(16, 16)] = a1
        out_v[out_row, pl.ds(32, 16)] = a2
        out_v[out_row, pl.ds(48, 16)] = a3

    def chunk_pass(c, cbuf):
        nxt = 1 - cbuf

        @pl.when(c + 1 < _NCHUNK)
        def _():
            pltpu.async_copy(x_hbm.at[pl.ds(base + (c + 1) * _CH, _CH)],
                             idx_v.at[nxt], sem_i)

        out0 = c * _CH

        def pair(g2, carry):
            s = 2 * g2
            issue_sample(cbuf, s + 1, 1)
            wait_sample(0)
            accumulate(0, out0 + s)
            issue_sample(cbuf, s + 2, 0)
            wait_sample(1)
            accumulate(1, out0 + s + 1)
            return carry

        lax.fori_loop(0, (_CH - 2) // 2, pair, 0)

        issue_sample(cbuf, _CH - 1, 1)
        wait_sample(0)
        accumulate(0, out0 + _CH - 2)

        @pl.when(c + 1 < _NCHUNK)
        def _():
            pltpu.make_async_copy(x_hbm.at[pl.ds(base, _CH)],
                                  idx_v.at[nxt], sem_i).wait()
            issue_sample(nxt, 0, 0)

        wait_sample(1)
        accumulate(1, out0 + _CH - 1)

    pltpu.sync_copy(x_hbm.at[pl.ds(base, _CH)], idx_v.at[0])
    issue_sample(0, 0, 0)

    def outer(c2, carry):
        chunk_pass(2 * c2, 0)
        chunk_pass(2 * c2 + 1, 1)
        return carry

    lax.fori_loop(0, _NCHUNK // 2, outer, 0)

    pltpu.sync_copy(out_v, pooled_hbm.at[pl.ds(base, _SPW)])


def _mm_body(p_ref, w_ref, b_ref, o_ref):
    o_ref[...] = (
        jnp.dot(p_ref[...] * (1.0 / _L), w_ref[...],
                preferred_element_type=jnp.float32)
        + b_ref[...])


_mm = pl.pallas_call(
    _mm_body,
    out_shape=jax.ShapeDtypeStruct((_B, _NLAB), jnp.float32),
    grid=(8,),
    in_specs=[
        pl.BlockSpec((_B // 8, _EMB), lambda i: (i, 0)),
        pl.BlockSpec((_EMB, _NLAB), lambda i: (0, 0)),
        pl.BlockSpec((1, _NLAB), lambda i: (0, 0)),
    ],
    out_specs=pl.BlockSpec((_B // 8, _NLAB), lambda i: (i, 0)),
)


def _pack_table(emb_table):
    tb = emb_table.astype(jnp.bfloat16)
    h0 = jnp.stack([tb[:, 0:16], tb[:, 16:32]], axis=-1).reshape(_VROWS, 32)
    h1 = jnp.stack([tb[:, 32:48], tb[:, 48:64]], axis=-1).reshape(_VROWS, 32)
    packed = jnp.concatenate([h0, h1], axis=1).reshape(_VROWS, _W32, 2)
    return lax.bitcast_convert_type(packed, jnp.int32)


def kernel(x, emb_table, fc_w, fc_b):
    x3 = x.reshape(_B, 2, _HALF)
    pooled = _sc_pool(x3, _pack_table(emb_table))
    return _mm(pooled, fc_w.T, fc_b.reshape(1, _NLAB))


# speedup vs baseline: 35.3047x; 1.3410x over previous
import functools

import jax
import jax.numpy as jnp
from jax import lax
from jax.experimental import pallas as pl
from jax.experimental.pallas import tpu as pltpu
from jax.experimental.pallas import tpu_sc as plsc

_B = 16384
_L = 200
_EMB = 64
_NLAB = 50
_VROWS = 100001
_W32 = _EMB // 2

_NC = 2
_NS = 16
_NW = _NC * _NS
_SPW = _B // _NW
_HALF = _L // 2
_CH = 32
_NCHUNK = _SPW // _CH

_mesh = plsc.VectorSubcoreMesh(
    core_axis_name="c", subcore_axis_name="s",
    num_cores=_NC, num_subcores=_NS)


@functools.partial(
    pl.kernel,
    out_type=jax.ShapeDtypeStruct((_B, _EMB), jnp.float32),
    mesh=_mesh,
    scratch_types=[
        pltpu.VMEM((2, _CH, 2, _HALF), jnp.int32),
        pltpu.VMEM((4, _L, _W32), jnp.int32),
        pltpu.VMEM((_SPW, _EMB), jnp.float32),
        pltpu.SemaphoreType.DMA,
        pltpu.SemaphoreType.DMA,
    ],
    compiler_params=pltpu.CompilerParams(use_tc_tiling_on_sc=False,
                                         needs_layout_passes=False),
)
def _sc_pool(x_hbm, table_hbm, pooled_hbm, idx_v, rows_v, out_v,
             sem_g, sem_i):
    wid = lax.axis_index("s") * _NC + lax.axis_index("c")
    base = wid * _SPW

    def issue_sample(cbuf, s_local, rbuf):
        pltpu.async_copy(table_hbm.at[idx_v.at[cbuf, s_local, 0]],
                         rows_v.at[rbuf, pl.ds(0, _HALF)], sem_g)
        pltpu.async_copy(table_hbm.at[idx_v.at[cbuf, s_local, 1]],
                         rows_v.at[rbuf, pl.ds(_HALF, _HALF)], sem_g)

    def wait_sample(rbuf):
        pltpu.make_async_copy(table_hbm.at[pl.ds(0, _L)],
                              rows_v.at[rbuf], sem_g).wait()

    hi_mask = jnp.int32(-65536)

    def accumulate(rbuf, out_row):
        def acc_body(r, accs):
            accs = list(accs)
            for j in range(4):
                row = 4 * r + j
                v0 = rows_v[rbuf, row, pl.ds(0, 16)]
                v1 = rows_v[rbuf, row, pl.ds(16, 16)]
                a0, a1, a2, a3 = accs[4 * j:4 * j + 4]
                accs[4 * j + 0] = a0 + plsc.bitcast(v0 << 16, jnp.float32)
                accs[4 * j + 1] = a1 + plsc.bitcast(v0 & hi_mask,
                                                    jnp.float32)
                accs[4 * j + 2] = a2 + plsc.bitcast(v1 << 16, jnp.float32)
                accs[4 * j + 3] = a3 + plsc.bitcast(v1 & hi_mask,
                                                    jnp.float32)
            return tuple(accs)

        z = jnp.zeros((16,), jnp.float32)
        accs = lax.fori_loop(0, _L // 4, acc_body, (z,) * 16)
        out_v[out_row, pl.ds(0, 16)] = accs[0] + accs[4] + accs[8] + accs[12]
        out_v[out_row, pl.ds(16, 16)] = accs[1] + accs[5] + accs[9] + accs[13]
        out_v[out_row, pl.ds(32, 16)] = accs[2] + accs[6] + accs[10] + accs[14]
        out_v[out_row, pl.ds(48, 16)] = accs[3] + accs[7] + accs[11] + accs[15]

    def chunk_pass(c, cbuf):
        nxt = 1 - cbuf

        @pl.when(c + 1 < _NCHUNK)
        def _():
            pltpu.async_copy(x_hbm.at[pl.ds(base + (c + 1) * _CH, _CH)],
                             idx_v.at[nxt], sem_i)

        out0 = c * _CH

        def quad(q, carry):
            s = 4 * q
            for b in range(4):
                issue_sample(cbuf, s + b + 3, (b + 3) % 4)
                wait_sample(b)
                accumulate(b, out0 + s + b)
            return carry

        lax.fori_loop(0, (_CH - 4) // 4, quad, 0)

        issue_sample(cbuf, _CH - 1, 3)
        wait_sample(0)
        accumulate(0, out0 + _CH - 4)

        @pl.when(c + 1 < _NCHUNK)
        def _():
            pltpu.make_async_copy(x_hbm.at[pl.ds(base, _CH)],
                                  idx_v.at[nxt], sem_i).wait()
            issue_sample(nxt, 0, 0)

        wait_sample(1)
        accumulate(1, out0 + _CH - 3)

        @pl.when(c + 1 < _NCHUNK)
        def _():
            issue_sample(nxt, 1, 1)

        wait_sample(2)
        accumulate(2, out0 + _CH - 2)

        @pl.when(c + 1 < _NCHUNK)
        def _():
            issue_sample(nxt, 2, 2)

        wait_sample(3)
        accumulate(3, out0 + _CH - 1)

    pltpu.sync_copy(x_hbm.at[pl.ds(base, _CH)], idx_v.at[0])
    issue_sample(0, 0, 0)
    issue_sample(0, 1, 1)
    issue_sample(0, 2, 2)

    def outer(c2, carry):
        chunk_pass(2 * c2, 0)
        chunk_pass(2 * c2 + 1, 1)
        return carry

    lax.fori_loop(0, _NCHUNK // 2, outer, 0)

    pltpu.sync_copy(out_v, pooled_hbm.at[pl.ds(base, _SPW)])


def _mm_body(p_ref, w_ref, b_ref, o_ref):
    o_ref[...] = (
        jnp.dot(p_ref[...] * (1.0 / _L), w_ref[...],
                preferred_element_type=jnp.float32)
        + b_ref[...])


_mm = pl.pallas_call(
    _mm_body,
    out_shape=jax.ShapeDtypeStruct((_B, _NLAB), jnp.float32),
    grid=(8,),
    in_specs=[
        pl.BlockSpec((_B // 8, _EMB), lambda i: (i, 0)),
        pl.BlockSpec((_EMB, _NLAB), lambda i: (0, 0)),
        pl.BlockSpec((1, _NLAB), lambda i: (0, 0)),
    ],
    out_specs=pl.BlockSpec((_B // 8, _NLAB), lambda i: (i, 0)),
)


def _pack_table(emb_table):
    tb = emb_table.astype(jnp.bfloat16)
    h0 = jnp.stack([tb[:, 0:16], tb[:, 16:32]], axis=-1).reshape(_VROWS, 32)
    h1 = jnp.stack([tb[:, 32:48], tb[:, 48:64]], axis=-1).reshape(_VROWS, 32)
    packed = jnp.concatenate([h0, h1], axis=1).reshape(_VROWS, _W32, 2)
    return lax.bitcast_convert_type(packed, jnp.int32)


def kernel(x, emb_table, fc_w, fc_b):
    x3 = x.reshape(_B, 2, _HALF)
    pooled = _sc_pool(x3, _pack_table(emb_table))
    return _mm(pooled, fc_w.T, fc_b.reshape(1, _NLAB))
